# baseline (speedup 1.0000x reference)
import functools

import jax
import jax.numpy as jnp
from jax.experimental import pallas as pl
from jax.experimental.pallas import tpu as pltpu

MD = 7
PATCH = 32
NUM_MA = 24


def _diff_kernel(x_ref, out_ref, ps_ref):
    c = pl.program_id(1)
    nc = pl.num_programs(1)
    T = x_ref.shape[2]

    rows = []
    for s in range(MD):
        sl = pl.ds(s * PATCH, PATCH)
        part = None
        prev = x_ref[0, 0, 0, sl, :]
        for t in range(1, T):
            cur = x_ref[0, 0, t, sl, :]
            d = jnp.abs(cur - prev)
            part = d if part is None else part + d
            prev = cur
        rows.append(jnp.concatenate(
            [part[:, j * PATCH:(j + 1) * PATCH].sum(axis=1, keepdims=True)
             for j in range(MD)], axis=1).sum(axis=0, keepdims=True))
    ps = jnp.concatenate(rows, axis=0)

    @pl.when(c == 0)
    def _():
        ps_ref[...] = ps

    @pl.when(c != 0)
    def _():
        ps_ref[...] = ps_ref[...] + ps

    @pl.when(c == nc - 1)
    def _():
        v = ps_ref[...]
        idx = jax.lax.broadcasted_iota(jnp.int32, (MD, MD), 0) * MD + \
              jax.lax.broadcasted_iota(jnp.int32, (MD, MD), 1)
        a = v[:, :, None, None]
        b = v[None, None, :, :]
        ia = idx[:, :, None, None]
        ib = idx[None, None, :, :]
        beats = (b > a) | ((b == a) & (ib < ia))
        rank = beats.astype(jnp.int32).sum(axis=(2, 3))
        out_ref[0] = (rank < NUM_MA).astype(jnp.float32)


@jax.jit
def kernel(x):
    B, C, T, H, W = x.shape
    return pl.pallas_call(
        _diff_kernel,
        grid=(B, C),
        in_specs=[pl.BlockSpec((1, 1, T, H, W), lambda b, c: (b, c, 0, 0, 0))],
        out_specs=pl.BlockSpec((1, MD, MD), lambda b, c: (b, 0, 0)),
        out_shape=jax.ShapeDtypeStruct((B, MD, MD), jnp.float32),
        scratch_shapes=[pltpu.VMEM((MD, MD), jnp.float32)],
    )(x)


# speedup vs baseline: 1.9271x; 1.9271x over previous
import functools

import jax
import jax.numpy as jnp
from jax.experimental import pallas as pl
from jax.experimental.pallas import tpu as pltpu

MD = 7
PATCH = 32
NUM_MA = 24


def _diff_kernel(x_ref, out_ref, acc_ref):
    c = pl.program_id(1)
    nc = pl.num_programs(1)
    T = x_ref.shape[2]
    H, W = x_ref.shape[3], x_ref.shape[4]

    parts = []
    for s in range(MD):
        lo, hi = s * PATCH, (s + 1) * PATCH
        part = None
        prev = x_ref[0, 0, 0, lo:hi, :]
        for t in range(1, T):
            cur = x_ref[0, 0, t, lo:hi, :]
            d = jnp.abs(cur - prev)
            part = d if part is None else part + d
            prev = cur
        parts.append(part)
    full = jnp.concatenate(parts, axis=0)

    @pl.when(c == 0)
    def _():
        acc_ref[...] = full

    @pl.when(c != 0)
    def _():
        acc_ref[...] = acc_ref[...] + full

    @pl.when(c == nc - 1)
    def _():
        acc = acc_ref[...]
        hg = jax.lax.broadcasted_iota(jnp.int32, (MD, H), 1) // PATCH
        gi = jax.lax.broadcasted_iota(jnp.int32, (MD, H), 0)
        G = (hg == gi).astype(jnp.float32)
        wg = jax.lax.broadcasted_iota(jnp.int32, (W, MD), 0) // PATCH
        gj = jax.lax.broadcasted_iota(jnp.int32, (W, MD), 1)
        P = (wg == gj).astype(jnp.float32)
        ps = jnp.dot(jnp.dot(G, acc, precision=jax.lax.Precision.HIGHEST,
                             preferred_element_type=jnp.float32),
                     P, precision=jax.lax.Precision.HIGHEST,
                     preferred_element_type=jnp.float32)

        idx = jax.lax.broadcasted_iota(jnp.int32, (MD, MD), 0) * MD + \
              jax.lax.broadcasted_iota(jnp.int32, (MD, MD), 1)
        a = ps[:, :, None, None]
        b = ps[None, None, :, :]
        ia = idx[:, :, None, None]
        ib = idx[None, None, :, :]
        beats = (b > a) | ((b == a) & (ib < ia))
        rank = beats.astype(jnp.int32).sum(axis=(2, 3))
        out_ref[0] = (rank < NUM_MA).astype(jnp.float32)


@jax.jit
def kernel(x):
    B, C, T, H, W = x.shape
    return pl.pallas_call(
        _diff_kernel,
        grid=(B, C),
        in_specs=[pl.BlockSpec((1, 1, T, H, W), lambda b, c: (b, c, 0, 0, 0))],
        out_specs=pl.BlockSpec((1, MD, MD), lambda b, c: (b, 0, 0)),
        out_shape=jax.ShapeDtypeStruct((B, MD, MD), jnp.float32),
        scratch_shapes=[pltpu.VMEM((H, W), jnp.float32)],
    )(x)
